# Initial kernel scaffold; baseline (speedup 1.0000x reference)
import jax
import jax.numpy as jnp
from jax.experimental import pallas as pl


def kernel(user_idx, context_idx, edge_index, user_emb, service_emb, W1, att_src1, att_dst1, b1, W2, att_src2, att_dst2, b2, Wfc, bfc):
    raise NotImplementedError("write your pallas kernel here")


# speedup vs baseline: 1.1617x; 1.1617x over previous
import functools

import jax
import jax.numpy as jnp
import numpy as np
from jax import lax
from jax.experimental import pallas as pl
from jax.experimental.pallas import tpu as pltpu

_HEADS = 2
_DIM = 32


def _fc_body(x_ref, w_ref, b_ref, o_ref):
    o_ref[...] = (
        jnp.dot(x_ref[...], w_ref[...], preferred_element_type=jnp.float32)
        + b_ref[...]
    )


def _final_fc(x, Wfc, bfc):
    B, K = x.shape
    N = Wfc.shape[1]
    NP = ((N + 127) // 128) * 128
    Wp = jnp.pad(Wfc, ((0, 0), (0, NP - N)))
    bp = jnp.pad(bfc, (0, NP - N)).reshape(1, NP)
    CB = 2176 if NP % 2176 == 0 else NP
    grid = (NP // CB,)
    out = pl.pallas_call(
        _fc_body,
        grid=grid,
        in_specs=[
            pl.BlockSpec((B, K), lambda j: (0, 0)),
            pl.BlockSpec((K, CB), lambda j: (0, j)),
            pl.BlockSpec((1, CB), lambda j: (0, j)),
        ],
        out_specs=pl.BlockSpec((B, CB), lambda j: (0, j)),
        out_shape=jax.ShapeDtypeStruct((B, NP), jnp.float32),
    )(x, Wp, bp)
    return out[:, :N]


def _gat_layer_jnp(x, src, dst, W, a_src, a_dst, b, heads, out_ch, n):
    h = (x @ W).reshape(n, heads, out_ch)
    asrc = (h * a_src[None, :, :]).sum(-1)
    adst = (h * a_dst[None, :, :]).sum(-1)
    e = asrc[src] + adst[dst]
    e = jnp.where(e > 0, e, 0.2 * e)
    s = jnp.maximum(asrc.max(0) + adst.max(0), 0.0)
    p = jnp.exp(e - s[None, :])
    denom = jax.ops.segment_sum(p, dst, num_segments=n)
    acc = jax.ops.segment_sum(h[src] * p[:, :, None], dst, num_segments=n)
    out = acc / denom[:, :, None]
    return out.reshape(n, heads * out_ch) + b


def kernel(user_idx, context_idx, edge_index, user_emb, service_emb, W1,
           att_src1, att_dst1, b1, W2, att_src2, att_dst2, b2, Wfc, bfc):
    n = service_emb.shape[0]
    sl = jnp.arange(n, dtype=edge_index.dtype)
    src = jnp.concatenate([edge_index[0], sl])
    dst = jnp.concatenate([edge_index[1], sl])
    user_vec = user_emb[user_idx]
    g1 = jax.nn.elu(_gat_layer_jnp(service_emb, src, dst, W1, att_src1,
                                   att_dst1, b1, _HEADS, _DIM, n))
    g2 = _gat_layer_jnp(g1, src, dst, W2, att_src2, att_dst2, b2, 1, _DIM, n)
    ctx = g2[context_idx].reshape(context_idx.shape[0], -1)
    x = jnp.concatenate([user_vec, ctx], axis=1)
    return _final_fc(x, Wfc, bfc)


# speedup vs baseline: 17.3595x; 14.9434x over previous
import functools

import jax
import jax.numpy as jnp
from jax import lax
from jax.experimental import pallas as pl
from jax.experimental.pallas import tpu as pltpu
import jax.experimental.pallas.tpu_sc as plsc

_N = 50000
_NQ = 4
_QR = 12544
_NP = _NQ * _QR
_ACC_R = _QR + 128
_D = 32
_H = 2
_B = 1024
_CTX = 3
_K = 128
_EP = 851968
_RZT = _ACC_R // 16
_RWT = _QR // 16


_RB = 6272


def _tc1_body(x_ref, w_ref, a_ref, h_ref, sa_ref, sm_ref):
    h = jnp.dot(x_ref[...], w_ref[...], preferred_element_type=jnp.float32)
    h_ref[0] = h[:, :_D]
    h_ref[1] = h[:, _D:]
    sa = lax.dot_general(a_ref[...], h, (((0,), (1,)), ((), ())),
                         preferred_element_type=jnp.float32)
    sa_ref[...] = sa
    blkmax = jnp.max(sa, axis=1, keepdims=True)
    i = pl.program_id(0)

    @pl.when(i == 0)
    def _():
        sm_ref[...] = blkmax

    @pl.when(i > 0)
    def _():
        sm_ref[...] = jnp.maximum(sm_ref[...], blkmax)


def _tc_prolog(x, W1, A1):
    return pl.pallas_call(
        _tc1_body,
        grid=(_NP // _RB,),
        in_specs=[
            pl.BlockSpec((_RB, _D), lambda i: (i, 0)),
            pl.BlockSpec((_D, 2 * _D), lambda i: (0, 0)),
            pl.BlockSpec((2 * _D, 4), lambda i: (0, 0)),
        ],
        out_specs=[
            pl.BlockSpec((2, _RB, _D), lambda i: (0, i, 0)),
            pl.BlockSpec((4, _RB), lambda i: (0, i)),
            pl.BlockSpec((4, 1), lambda i: (0, 0)),
        ],
        out_shape=[
            jax.ShapeDtypeStruct((2, _NP, _D), jnp.float32),
            jax.ShapeDtypeStruct((4, _NP), jnp.float32),
            jax.ShapeDtypeStruct((4, 1), jnp.float32),
        ],
    )(x, W1, A1)


def _tc2_body(acc_ref, den_ref, b1_ref, w2a_ref, w2b_ref, a2_ref,
              h2_ref, sa2_ref, sm2_ref):
    def _elu(x):
        return jnp.where(x > 0, x, jnp.exp(jnp.minimum(x, 0.0)) - 1.0)
    g0 = _elu(acc_ref[0] / (den_ref[0] + 1e-30) + b1_ref[0, :_D])
    g1 = _elu(acc_ref[1] / (den_ref[1] + 1e-30) + b1_ref[0, _D:])
    h2 = (jnp.dot(g0, w2a_ref[...], preferred_element_type=jnp.float32)
          + jnp.dot(g1, w2b_ref[...], preferred_element_type=jnp.float32))
    h2_ref[...] = h2
    sa2 = lax.dot_general(a2_ref[...], h2, (((0,), (1,)), ((), ())),
                          preferred_element_type=jnp.float32)
    sa2_ref[...] = sa2
    blkmax = jnp.max(sa2, axis=1, keepdims=True)
    i = pl.program_id(0)

    @pl.when(i == 0)
    def _():
        sm2_ref[...] = blkmax

    @pl.when(i > 0)
    def _():
        sm2_ref[...] = jnp.maximum(sm2_ref[...], blkmax)


def _tc_mid(acc1, den1, b1, W2a, W2b, A2):
    return pl.pallas_call(
        _tc2_body,
        grid=(_NP // _RB,),
        in_specs=[
            pl.BlockSpec((2, _RB, _D), lambda i: (0, i, 0)),
            pl.BlockSpec((2, _RB, 1), lambda i: (0, i, 0)),
            pl.BlockSpec((1, 2 * _D), lambda i: (0, 0)),
            pl.BlockSpec((_D, _D), lambda i: (0, 0)),
            pl.BlockSpec((_D, _D), lambda i: (0, 0)),
            pl.BlockSpec((_D, 4), lambda i: (0, 0)),
        ],
        out_specs=[
            pl.BlockSpec((_RB, _D), lambda i: (i, 0)),
            pl.BlockSpec((4, _RB), lambda i: (0, i)),
            pl.BlockSpec((4, 1), lambda i: (0, 0)),
        ],
        out_shape=[
            jax.ShapeDtypeStruct((_NP, _D), jnp.float32),
            jax.ShapeDtypeStruct((4, _NP), jnp.float32),
            jax.ShapeDtypeStruct((4, 1), jnp.float32),
        ],
    )(acc1, den1, b1, W2a, W2b, A2)


def _tc3_body(ctx_ref, den_ref, b2_ref, out_ref):
    out_ref[...] = ctx_ref[...] / (den_ref[...] + 1e-30) + b2_ref[...]


def _tc_ctxnorm(ctx_rows, den_ctx, b2):
    return pl.pallas_call(
        _tc3_body,
        out_shape=jax.ShapeDtypeStruct((_B * _CTX, _D), jnp.float32),
    )(ctx_rows, den_ctx, b2)


def _fc_body(x_ref, w_ref, b_ref, o_ref):
    o_ref[...] = (
        jnp.dot(x_ref[...], w_ref[...], preferred_element_type=jnp.float32)
        + b_ref[...]
    )


def _final_fc(x, Wfc, bfc):
    B, Kd = x.shape
    Nout = Wfc.shape[1]
    NPAD = ((Nout + 127) // 128) * 128
    Wp = jnp.pad(Wfc, ((0, 0), (0, NPAD - Nout)))
    bp = jnp.pad(bfc, (0, NPAD - Nout)).reshape(1, NPAD)
    CB = 2176 if NPAD % 2176 == 0 else NPAD
    grid = (NPAD // CB,)
    out = pl.pallas_call(
        _fc_body,
        grid=grid,
        in_specs=[
            pl.BlockSpec((B, Kd), lambda j: (0, 0)),
            pl.BlockSpec((Kd, CB), lambda j: (0, j)),
            pl.BlockSpec((1, CB), lambda j: (0, j)),
        ],
        out_specs=pl.BlockSpec((B, CB), lambda j: (0, j)),
        out_shape=jax.ShapeDtypeStruct((B, NPAD), jnp.float32),
    )(x, Wp, bp)
    return out[:, :Nout]


def _edge_body(split, row_off_mult,
               h_hbm, src_hbm, dst_hbm, sa_hbm, shift_hbm,
               acc_out, den_out,
               asrc_l, adst_l, idx_s, idx_d, rows_v, p_v, shift_v,
               acc_s, den_s, sem):
    c = lax.axis_index("c")
    s = lax.axis_index("s")
    zero16 = jnp.zeros((16,), jnp.float32)

    pltpu.sync_copy(sa_hbm.at[c], asrc_l)
    pltpu.sync_copy(sa_hbm.at[2 + c], adst_l)
    pltpu.sync_copy(shift_hbm.at[c], shift_v)
    s_vec = shift_v[...]

    if split:
        per_tile = _EP // 32
        base0 = (c * 16 + s) * per_tile
    else:
        per_tile = _EP // 16
        base0 = s * per_tile
    row_off = c * (_NP * row_off_mult)

    for q in range(_NQ):
        def _zrow(j, _):
            rows_v[j, pl.ds(0, 16)] = zero16
            rows_v[j, pl.ds(16, 16)] = zero16
            return 0
        lax.fori_loop(0, _K, _zrow, 0)

        def _zp(i, _):
            p_v[pl.ds(i * 16, 16)] = zero16
            return 0
        lax.fori_loop(0, _K // 16, _zp, 0)

        rz = s * _RZT
        nfull = _RZT // _K
        tail = _RZT - nfull * _K
        for j in range(nfull):
            pltpu.sync_copy(rows_v, acc_s.at[pl.ds(rz + j * _K, _K)])
            pltpu.sync_copy(p_v, den_s.at[pl.ds(rz + j * _K, _K)])
        if tail:
            pltpu.sync_copy(rows_v.at[pl.ds(0, tail)],
                            acc_s.at[pl.ds(rz + nfull * _K, tail)])
            pltpu.sync_copy(p_v.at[pl.ds(0, tail)],
                            den_s.at[pl.ds(rz + nfull * _K, tail)])
        plsc.subcore_barrier()

        def _chunk(g, _):
            b = base0 + g * _K
            pltpu.sync_copy(src_hbm.at[pl.ds(b, _K)], idx_s)
            pltpu.sync_copy(dst_hbm.at[pl.ds(b, _K)], idx_d)
            for i in range(_K // 16):
                vs = idx_s[pl.ds(i * 16, 16)]
                vd = idx_d[pl.ds(i * 16, 16)]
                a_s = plsc.load_gather(asrc_l, [vs])
                a_d = plsc.load_gather(adst_l, [vd])
                e = a_s + a_d
                e = jnp.where(e > 0, e, 0.2 * e)
                p_v[pl.ds(i * 16, 16)] = jnp.exp(e - s_vec)
                vq = vd - (q * _QR)
                ok = (vq >= 0) & (vq < _QR)
                idx_d[pl.ds(i * 16, 16)] = jnp.where(ok, vq, _QR)
                if row_off_mult:
                    idx_s[pl.ds(i * 16, 16)] = vs + row_off
            pltpu.async_copy(h_hbm.at[idx_s], rows_v, sem).wait()

            def _scale(i, _):
                j = i * 16
                p16 = p_v[pl.ds(j, 16)]
                for u in range(16):
                    ps = p16[u]
                    rows_v[j + u, pl.ds(0, 16)] = (
                        rows_v[j + u, pl.ds(0, 16)] * ps)
                    rows_v[j + u, pl.ds(16, 16)] = (
                        rows_v[j + u, pl.ds(16, 16)] * ps)
                return 0
            lax.fori_loop(0, _K // 16, _scale, 0)

            pltpu.sync_copy(rows_v, acc_s.at[idx_d], add=True)
            pltpu.sync_copy(p_v, den_s.at[idx_d], add=True)
            return 0

        lax.fori_loop(0, per_tile // _K, _chunk, 0)
        plsc.subcore_barrier()

        rw = s * _RWT
        out_base = c * _NP + q * _QR
        pltpu.sync_copy(acc_s.at[pl.ds(rw, _RWT)],
                        acc_out.at[pl.ds(out_base + rw, _RWT)])
        pltpu.sync_copy(den_s.at[pl.ds(rw, _RWT)],
                        den_out.at[pl.ds(out_base + rw, _RWT)])
        plsc.subcore_barrier()


def _make_edge_kernel(split, row_off_mult):
    body = functools.partial(_edge_body, split, row_off_mult)
    return pl.kernel(
        body,
        out_type=[
            jax.ShapeDtypeStruct((2 * _NP, _D), jnp.float32),
            jax.ShapeDtypeStruct((2 * _NP,), jnp.float32),
        ],
        mesh=plsc.VectorSubcoreMesh(core_axis_name="c", subcore_axis_name="s",
                                    num_cores=2, num_subcores=16),
        compiler_params=pltpu.CompilerParams(needs_layout_passes=False, use_tc_tiling_on_sc=False),
        scratch_types=[
            pltpu.VMEM((_NP,), jnp.float32),
            pltpu.VMEM((_NP,), jnp.float32),
            pltpu.VMEM((_K,), jnp.int32),
            pltpu.VMEM((_K,), jnp.int32),
            pltpu.VMEM((_K, _D), jnp.float32),
            pltpu.VMEM((_K,), jnp.float32),
            pltpu.VMEM((16,), jnp.float32),
            pltpu.VMEM_SHARED((_ACC_R, _D), jnp.float32),
            pltpu.VMEM_SHARED((_ACC_R,), jnp.float32),
            pltpu.SemaphoreType.DMA,
        ],
    )


def _gather_body(uemb_hbm, uidx_hbm, cidx_hbm, acc2_hbm, den2_hbm,
                 urows_out, crows_out, dctx_out,
                 den_l, uidx_v, cidxa_v, cidxb_v, urows_v, ca_v, cb_v,
                 dsum_v, sem):
    c = lax.axis_index("c")
    s = lax.axis_index("s")
    wid = c * 16 + s

    upw = _B // 32
    ub = wid * upw
    pltpu.sync_copy(uidx_hbm.at[pl.ds(ub, upw)], uidx_v)
    pltpu.async_copy(uemb_hbm.at[uidx_v], urows_v, sem).wait()
    pltpu.sync_copy(urows_v, urows_out.at[pl.ds(ub, upw)])

    pltpu.sync_copy(den2_hbm, den_l)

    cpw = (_B * _CTX) // 32
    cb = wid * cpw
    pltpu.sync_copy(cidx_hbm.at[pl.ds(cb, cpw)], cidxa_v)
    for i in range(cpw // 16):
        cidxb_v[pl.ds(i * 16, 16)] = cidxa_v[pl.ds(i * 16, 16)] + _NP
    pltpu.async_copy(acc2_hbm.at[cidxa_v], ca_v, sem).wait()
    pltpu.async_copy(acc2_hbm.at[cidxb_v], cb_v, sem).wait()

    def _sum(j, _):
        ca_v[j, pl.ds(0, 16)] = ca_v[j, pl.ds(0, 16)] + cb_v[j, pl.ds(0, 16)]
        ca_v[j, pl.ds(16, 16)] = ca_v[j, pl.ds(16, 16)] + cb_v[j, pl.ds(16, 16)]
        return 0
    lax.fori_loop(0, cpw, _sum, 0)
    pltpu.sync_copy(ca_v, crows_out.at[pl.ds(cb, cpw)])

    for i in range(cpw // 16):
        va = cidxa_v[pl.ds(i * 16, 16)]
        vb = cidxb_v[pl.ds(i * 16, 16)]
        da = plsc.load_gather(den_l, [va])
        db = plsc.load_gather(den_l, [vb])
        dsum_v[pl.ds(i * 16, 16)] = da + db
    pltpu.sync_copy(dsum_v, dctx_out.at[pl.ds(cb, cpw)])


def _make_gather_kernel():
    return pl.kernel(
        _gather_body,
        out_type=[
            jax.ShapeDtypeStruct((_B, _D), jnp.float32),
            jax.ShapeDtypeStruct((_B * _CTX, _D), jnp.float32),
            jax.ShapeDtypeStruct((_B * _CTX,), jnp.float32),
        ],
        mesh=plsc.VectorSubcoreMesh(core_axis_name="c", subcore_axis_name="s",
                                    num_cores=2, num_subcores=16),
        compiler_params=pltpu.CompilerParams(needs_layout_passes=False, use_tc_tiling_on_sc=False),
        scratch_types=[
            pltpu.VMEM((2 * _NP,), jnp.float32),
            pltpu.VMEM((_B // 32,), jnp.int32),
            pltpu.VMEM(((_B * _CTX) // 32,), jnp.int32),
            pltpu.VMEM(((_B * _CTX) // 32,), jnp.int32),
            pltpu.VMEM((_B // 32, _D), jnp.float32),
            pltpu.VMEM(((_B * _CTX) // 32, _D), jnp.float32),
            pltpu.VMEM(((_B * _CTX) // 32, _D), jnp.float32),
            pltpu.VMEM(((_B * _CTX) // 32,), jnp.float32),
            pltpu.SemaphoreType.DMA,
        ],
    )


def kernel(user_idx, context_idx, edge_index, user_emb, service_emb, W1,
           att_src1, att_dst1, b1, W2, att_src2, att_dst2, b2, Wfc, bfc):
    f32 = jnp.float32
    sl = jnp.arange(_N, dtype=jnp.int32)
    pad_e = _EP - (edge_index.shape[1] + _N)
    src = jnp.concatenate([edge_index[0].astype(jnp.int32), sl,
                           jnp.zeros((pad_e,), jnp.int32)])
    dst = jnp.concatenate([edge_index[1].astype(jnp.int32), sl,
                           jnp.full((pad_e,), _N, jnp.int32)])
    nch = _EP // _K
    order = jnp.argsort(dst)
    src = src[order].reshape(_K, nch).T.reshape(-1)
    dst = dst[order].reshape(_K, nch).T.reshape(-1)

    x_pad = jnp.pad(service_emb, ((0, _NP - _N), (0, 0)))
    z = jnp.zeros((_D,), f32)
    A1 = jnp.stack([
        jnp.concatenate([att_src1[0], z]),
        jnp.concatenate([z, att_src1[1]]),
        jnp.concatenate([att_dst1[0], z]),
        jnp.concatenate([z, att_dst1[1]]),
    ], axis=1)
    A2 = jnp.stack([att_src2[0], att_src2[0],
                    att_dst2[0], att_dst2[0]], axis=1)

    h1_heads, sa1, sm1 = _tc_prolog(x_pad, W1, A1)
    sm1 = sm1[:, 0]
    shift1 = jnp.maximum(sm1[:2] + sm1[2:], 0.0)
    shift1_16 = jnp.broadcast_to(shift1[:, None], (2, 16))

    acc1, den1 = _make_edge_kernel(split=False, row_off_mult=1)(
        h1_heads.reshape(2 * _NP, _D), src, dst, sa1, shift1_16)

    h2, sa2, sm2 = _tc_mid(acc1.reshape(2, _NP, _D),
                           den1.reshape(2, _NP, 1),
                           b1.reshape(1, 2 * _D),
                           W2[:_D], W2[_D:], A2)
    sm2 = sm2[:, 0]
    shift2 = jnp.maximum(sm2[0] + sm2[2], 0.0)
    shift2_16 = jnp.broadcast_to(shift2, (2, 16))

    acc2, den2 = _make_edge_kernel(split=True, row_off_mult=0)(
        h2, src, dst, sa2, shift2_16)

    user_rows, ctx_rows, den_ctx = _make_gather_kernel()(
        user_emb, user_idx.astype(jnp.int32),
        context_idx.reshape(-1).astype(jnp.int32), acc2, den2)

    ctx_norm = _tc_ctxnorm(ctx_rows, den_ctx.reshape(_B * _CTX, 1),
                           b2.reshape(1, _D))
    x = jnp.concatenate([user_rows, ctx_norm.reshape(_B, _CTX * _D)], axis=1)
    return _final_fc(x, Wfc, bfc)


# speedup vs baseline: 17.6404x; 1.0162x over previous
import functools

import jax
import jax.numpy as jnp
from jax import lax
from jax.experimental import pallas as pl
from jax.experimental.pallas import tpu as pltpu
import jax.experimental.pallas.tpu_sc as plsc

_N = 50000
_NQ = 4
_QR = 12544
_NP = _NQ * _QR
_ACC_R = _QR + 128
_D = 32
_H = 2
_B = 1024
_CTX = 3
_K = 128
_EP = 851968
_RZT = _ACC_R // 16
_RWT = _QR // 16


_RB = 6272


def _tc1_body(x_ref, w_ref, a_ref, h_ref, sa_ref, sm_ref):
    h = jnp.dot(x_ref[...], w_ref[...], preferred_element_type=jnp.float32)
    h_ref[0] = h[:, :_D]
    h_ref[1] = h[:, _D:]
    sa = lax.dot_general(a_ref[...], h, (((0,), (1,)), ((), ())),
                         preferred_element_type=jnp.float32)
    sa_ref[...] = sa
    blkmax = jnp.max(sa, axis=1, keepdims=True)
    i = pl.program_id(0)

    @pl.when(i == 0)
    def _():
        sm_ref[...] = blkmax

    @pl.when(i > 0)
    def _():
        sm_ref[...] = jnp.maximum(sm_ref[...], blkmax)


def _tc_prolog(x, W1, A1):
    return pl.pallas_call(
        _tc1_body,
        grid=(_NP // _RB,),
        in_specs=[
            pl.BlockSpec((_RB, _D), lambda i: (i, 0)),
            pl.BlockSpec((_D, 2 * _D), lambda i: (0, 0)),
            pl.BlockSpec((2 * _D, 4), lambda i: (0, 0)),
        ],
        out_specs=[
            pl.BlockSpec((2, _RB, _D), lambda i: (0, i, 0)),
            pl.BlockSpec((4, _RB), lambda i: (0, i)),
            pl.BlockSpec((4, 1), lambda i: (0, 0)),
        ],
        out_shape=[
            jax.ShapeDtypeStruct((2, _NP, _D), jnp.float32),
            jax.ShapeDtypeStruct((4, _NP), jnp.float32),
            jax.ShapeDtypeStruct((4, 1), jnp.float32),
        ],
    )(x, W1, A1)


def _tc2_body(acc_ref, den_ref, b1_ref, w2a_ref, w2b_ref, a2_ref,
              h2_ref, sa2_ref, sm2_ref):
    def _elu(x):
        return jnp.where(x > 0, x, jnp.exp(jnp.minimum(x, 0.0)) - 1.0)
    g0 = _elu(acc_ref[0] / (den_ref[0] + 1e-30) + b1_ref[0, :_D])
    g1 = _elu(acc_ref[1] / (den_ref[1] + 1e-30) + b1_ref[0, _D:])
    h2 = (jnp.dot(g0, w2a_ref[...], preferred_element_type=jnp.float32)
          + jnp.dot(g1, w2b_ref[...], preferred_element_type=jnp.float32))
    h2_ref[...] = h2
    sa2 = lax.dot_general(a2_ref[...], h2, (((0,), (1,)), ((), ())),
                          preferred_element_type=jnp.float32)
    sa2_ref[...] = sa2
    blkmax = jnp.max(sa2, axis=1, keepdims=True)
    i = pl.program_id(0)

    @pl.when(i == 0)
    def _():
        sm2_ref[...] = blkmax

    @pl.when(i > 0)
    def _():
        sm2_ref[...] = jnp.maximum(sm2_ref[...], blkmax)


def _tc_mid(acc1, den1, b1, W2a, W2b, A2):
    return pl.pallas_call(
        _tc2_body,
        grid=(_NP // _RB,),
        in_specs=[
            pl.BlockSpec((2, _RB, _D), lambda i: (0, i, 0)),
            pl.BlockSpec((2, _RB, 1), lambda i: (0, i, 0)),
            pl.BlockSpec((1, 2 * _D), lambda i: (0, 0)),
            pl.BlockSpec((_D, _D), lambda i: (0, 0)),
            pl.BlockSpec((_D, _D), lambda i: (0, 0)),
            pl.BlockSpec((_D, 4), lambda i: (0, 0)),
        ],
        out_specs=[
            pl.BlockSpec((_RB, _D), lambda i: (i, 0)),
            pl.BlockSpec((4, _RB), lambda i: (0, i)),
            pl.BlockSpec((4, 1), lambda i: (0, 0)),
        ],
        out_shape=[
            jax.ShapeDtypeStruct((_NP, _D), jnp.float32),
            jax.ShapeDtypeStruct((4, _NP), jnp.float32),
            jax.ShapeDtypeStruct((4, 1), jnp.float32),
        ],
    )(acc1, den1, b1, W2a, W2b, A2)


def _tc3_body(ctx_ref, den_ref, b2_ref, out_ref):
    out_ref[...] = ctx_ref[...] / (den_ref[...] + 1e-30) + b2_ref[...]


def _tc_ctxnorm(ctx_rows, den_ctx, b2):
    return pl.pallas_call(
        _tc3_body,
        out_shape=jax.ShapeDtypeStruct((_B * _CTX, _D), jnp.float32),
    )(ctx_rows, den_ctx, b2)


def _fc_body(x_ref, w_ref, b_ref, o_ref):
    o_ref[...] = (
        jnp.dot(x_ref[...], w_ref[...], preferred_element_type=jnp.float32)
        + b_ref[...]
    )


def _final_fc(x, Wfc, bfc):
    B, Kd = x.shape
    Nout = Wfc.shape[1]
    NPAD = ((Nout + 127) // 128) * 128
    Wp = jnp.pad(Wfc, ((0, 0), (0, NPAD - Nout)))
    bp = jnp.pad(bfc, (0, NPAD - Nout)).reshape(1, NPAD)
    CB = 2176 if NPAD % 2176 == 0 else NPAD
    grid = (NPAD // CB,)
    out = pl.pallas_call(
        _fc_body,
        grid=grid,
        in_specs=[
            pl.BlockSpec((B, Kd), lambda j: (0, 0)),
            pl.BlockSpec((Kd, CB), lambda j: (0, j)),
            pl.BlockSpec((1, CB), lambda j: (0, j)),
        ],
        out_specs=pl.BlockSpec((B, CB), lambda j: (0, j)),
        out_shape=jax.ShapeDtypeStruct((B, NPAD), jnp.float32),
    )(x, Wp, bp)
    return out[:, :Nout]


def _edge_body(split, row_off_mult,
               h_hbm, src_hbm, dst_hbm, sa_hbm, shift_hbm,
               acc_out, den_out, rowsbuf, pbuf,
               asrc_l, adst_l, idx_s, idx_d, rows_v, p_v, shift_v,
               acc_s, den_s, sem):
    c = lax.axis_index("c")
    s = lax.axis_index("s")
    zero16 = jnp.zeros((16,), jnp.float32)

    pltpu.sync_copy(sa_hbm.at[c], asrc_l)
    pltpu.sync_copy(sa_hbm.at[2 + c], adst_l)
    pltpu.sync_copy(shift_hbm.at[c], shift_v)
    s_vec = shift_v[...]

    if split:
        per_tile = _EP // 32
        base0 = (c * 16 + s) * per_tile
    else:
        per_tile = _EP // 16
        base0 = s * per_tile
    row_off = c * (_NP * row_off_mult)

    for q in range(_NQ):
        def _zrow(j, _):
            rows_v[j, pl.ds(0, 16)] = zero16
            rows_v[j, pl.ds(16, 16)] = zero16
            return 0
        lax.fori_loop(0, _K, _zrow, 0)

        def _zp(i, _):
            p_v[pl.ds(i * 16, 16)] = zero16
            return 0
        lax.fori_loop(0, _K // 16, _zp, 0)

        rz = s * _RZT
        nfull = _RZT // _K
        tail = _RZT - nfull * _K
        for j in range(nfull):
            pltpu.sync_copy(rows_v, acc_s.at[pl.ds(rz + j * _K, _K)])
            pltpu.sync_copy(p_v, den_s.at[pl.ds(rz + j * _K, _K)])
        if tail:
            pltpu.sync_copy(rows_v.at[pl.ds(0, tail)],
                            acc_s.at[pl.ds(rz + nfull * _K, tail)])
            pltpu.sync_copy(p_v.at[pl.ds(0, tail)],
                            den_s.at[pl.ds(rz + nfull * _K, tail)])
        plsc.subcore_barrier()

        if q == 0:
            def _chunk(g, _):
                b = base0 + g * _K
                pltpu.sync_copy(src_hbm.at[pl.ds(b, _K)], idx_s)
                pltpu.sync_copy(dst_hbm.at[pl.ds(b, _K)], idx_d)
                for i in range(_K // 16):
                    vs = idx_s[pl.ds(i * 16, 16)]
                    vd = idx_d[pl.ds(i * 16, 16)]
                    a_s = plsc.load_gather(asrc_l, [vs])
                    a_d = plsc.load_gather(adst_l, [vd])
                    e = a_s + a_d
                    e = jnp.where(e > 0, e, 0.2 * e)
                    p_v[pl.ds(i * 16, 16)] = jnp.exp(e - s_vec)
                    ok = vd < _QR
                    idx_d[pl.ds(i * 16, 16)] = jnp.where(ok, vd, _QR)
                    if row_off_mult:
                        idx_s[pl.ds(i * 16, 16)] = vs + row_off
                pltpu.async_copy(h_hbm.at[idx_s], rows_v, sem).wait()

                def _scale(i, _):
                    j = i * 16
                    p16 = p_v[pl.ds(j, 16)]
                    for u in range(16):
                        ps = p16[u]
                        rows_v[j + u, pl.ds(0, 16)] = (
                            rows_v[j + u, pl.ds(0, 16)] * ps)
                        rows_v[j + u, pl.ds(16, 16)] = (
                            rows_v[j + u, pl.ds(16, 16)] * ps)
                    return 0
                lax.fori_loop(0, _K // 16, _scale, 0)

                pltpu.sync_copy(rows_v, rowsbuf.at[pl.ds(c * _EP + b, _K)])
                pltpu.sync_copy(p_v, pbuf.at[pl.ds(c * _EP + b, _K)])
                pltpu.sync_copy(rows_v, acc_s.at[idx_d], add=True)
                pltpu.sync_copy(p_v, den_s.at[idx_d], add=True)
                return 0
        else:
            def _chunk(g, _):
                b = base0 + g * _K
                pltpu.sync_copy(dst_hbm.at[pl.ds(b, _K)], idx_d)
                pltpu.sync_copy(rowsbuf.at[pl.ds(c * _EP + b, _K)], rows_v)
                pltpu.sync_copy(pbuf.at[pl.ds(c * _EP + b, _K)], p_v)
                for i in range(_K // 16):
                    vd = idx_d[pl.ds(i * 16, 16)]
                    vq = vd - (q * _QR)
                    ok = (vq >= 0) & (vq < _QR)
                    idx_d[pl.ds(i * 16, 16)] = jnp.where(ok, vq, _QR)
                pltpu.sync_copy(rows_v, acc_s.at[idx_d], add=True)
                pltpu.sync_copy(p_v, den_s.at[idx_d], add=True)
                return 0

        lax.fori_loop(0, per_tile // _K, _chunk, 0)
        plsc.subcore_barrier()

        rw = s * _RWT
        out_base = c * _NP + q * _QR
        pltpu.sync_copy(acc_s.at[pl.ds(rw, _RWT)],
                        acc_out.at[pl.ds(out_base + rw, _RWT)])
        pltpu.sync_copy(den_s.at[pl.ds(rw, _RWT)],
                        den_out.at[pl.ds(out_base + rw, _RWT)])
        plsc.subcore_barrier()


def _make_edge_kernel(split, row_off_mult):
    body = functools.partial(_edge_body, split, row_off_mult)
    return pl.kernel(
        body,
        out_type=[
            jax.ShapeDtypeStruct((2 * _NP, _D), jnp.float32),
            jax.ShapeDtypeStruct((2 * _NP,), jnp.float32),
            jax.ShapeDtypeStruct((2 * _EP, _D), jnp.float32),
            jax.ShapeDtypeStruct((2 * _EP,), jnp.float32),
        ],
        mesh=plsc.VectorSubcoreMesh(core_axis_name="c", subcore_axis_name="s",
                                    num_cores=2, num_subcores=16),
        compiler_params=pltpu.CompilerParams(needs_layout_passes=False, use_tc_tiling_on_sc=False),
        scratch_types=[
            pltpu.VMEM((_NP,), jnp.float32),
            pltpu.VMEM((_NP,), jnp.float32),
            pltpu.VMEM((_K,), jnp.int32),
            pltpu.VMEM((_K,), jnp.int32),
            pltpu.VMEM((_K, _D), jnp.float32),
            pltpu.VMEM((_K,), jnp.float32),
            pltpu.VMEM((16,), jnp.float32),
            pltpu.VMEM_SHARED((_ACC_R, _D), jnp.float32),
            pltpu.VMEM_SHARED((_ACC_R,), jnp.float32),
            pltpu.SemaphoreType.DMA,
        ],
    )


def _gather_body(uemb_hbm, uidx_hbm, cidx_hbm, acc2_hbm, den2_hbm,
                 urows_out, crows_out, dctx_out,
                 den_l, uidx_v, cidxa_v, cidxb_v, urows_v, ca_v, cb_v,
                 dsum_v, sem):
    c = lax.axis_index("c")
    s = lax.axis_index("s")
    wid = c * 16 + s

    upw = _B // 32
    ub = wid * upw
    pltpu.sync_copy(uidx_hbm.at[pl.ds(ub, upw)], uidx_v)
    pltpu.async_copy(uemb_hbm.at[uidx_v], urows_v, sem).wait()
    pltpu.sync_copy(urows_v, urows_out.at[pl.ds(ub, upw)])

    pltpu.sync_copy(den2_hbm, den_l)

    cpw = (_B * _CTX) // 32
    cb = wid * cpw
    pltpu.sync_copy(cidx_hbm.at[pl.ds(cb, cpw)], cidxa_v)
    for i in range(cpw // 16):
        cidxb_v[pl.ds(i * 16, 16)] = cidxa_v[pl.ds(i * 16, 16)] + _NP
    pltpu.async_copy(acc2_hbm.at[cidxa_v], ca_v, sem).wait()
    pltpu.async_copy(acc2_hbm.at[cidxb_v], cb_v, sem).wait()

    def _sum(j, _):
        ca_v[j, pl.ds(0, 16)] = ca_v[j, pl.ds(0, 16)] + cb_v[j, pl.ds(0, 16)]
        ca_v[j, pl.ds(16, 16)] = ca_v[j, pl.ds(16, 16)] + cb_v[j, pl.ds(16, 16)]
        return 0
    lax.fori_loop(0, cpw, _sum, 0)
    pltpu.sync_copy(ca_v, crows_out.at[pl.ds(cb, cpw)])

    for i in range(cpw // 16):
        va = cidxa_v[pl.ds(i * 16, 16)]
        vb = cidxb_v[pl.ds(i * 16, 16)]
        da = plsc.load_gather(den_l, [va])
        db = plsc.load_gather(den_l, [vb])
        dsum_v[pl.ds(i * 16, 16)] = da + db
    pltpu.sync_copy(dsum_v, dctx_out.at[pl.ds(cb, cpw)])


def _make_gather_kernel():
    return pl.kernel(
        _gather_body,
        out_type=[
            jax.ShapeDtypeStruct((_B, _D), jnp.float32),
            jax.ShapeDtypeStruct((_B * _CTX, _D), jnp.float32),
            jax.ShapeDtypeStruct((_B * _CTX,), jnp.float32),
        ],
        mesh=plsc.VectorSubcoreMesh(core_axis_name="c", subcore_axis_name="s",
                                    num_cores=2, num_subcores=16),
        compiler_params=pltpu.CompilerParams(needs_layout_passes=False, use_tc_tiling_on_sc=False),
        scratch_types=[
            pltpu.VMEM((2 * _NP,), jnp.float32),
            pltpu.VMEM((_B // 32,), jnp.int32),
            pltpu.VMEM(((_B * _CTX) // 32,), jnp.int32),
            pltpu.VMEM(((_B * _CTX) // 32,), jnp.int32),
            pltpu.VMEM((_B // 32, _D), jnp.float32),
            pltpu.VMEM(((_B * _CTX) // 32, _D), jnp.float32),
            pltpu.VMEM(((_B * _CTX) // 32, _D), jnp.float32),
            pltpu.VMEM(((_B * _CTX) // 32,), jnp.float32),
            pltpu.SemaphoreType.DMA,
        ],
    )


def kernel(user_idx, context_idx, edge_index, user_emb, service_emb, W1,
           att_src1, att_dst1, b1, W2, att_src2, att_dst2, b2, Wfc, bfc):
    f32 = jnp.float32
    sl = jnp.arange(_N, dtype=jnp.int32)
    pad_e = _EP - (edge_index.shape[1] + _N)
    src = jnp.concatenate([edge_index[0].astype(jnp.int32), sl,
                           jnp.zeros((pad_e,), jnp.int32)])
    dst = jnp.concatenate([edge_index[1].astype(jnp.int32), sl,
                           jnp.full((pad_e,), _N, jnp.int32)])
    nch = _EP // _K
    order = jnp.argsort(dst)
    src = src[order].reshape(_K, nch).T.reshape(-1)
    dst = dst[order].reshape(_K, nch).T.reshape(-1)

    x_pad = jnp.pad(service_emb, ((0, _NP - _N), (0, 0)))
    z = jnp.zeros((_D,), f32)
    A1 = jnp.stack([
        jnp.concatenate([att_src1[0], z]),
        jnp.concatenate([z, att_src1[1]]),
        jnp.concatenate([att_dst1[0], z]),
        jnp.concatenate([z, att_dst1[1]]),
    ], axis=1)
    A2 = jnp.stack([att_src2[0], att_src2[0],
                    att_dst2[0], att_dst2[0]], axis=1)

    h1_heads, sa1, sm1 = _tc_prolog(x_pad, W1, A1)
    sm1 = sm1[:, 0]
    shift1 = jnp.maximum(sm1[:2] + sm1[2:], 0.0)
    shift1_16 = jnp.broadcast_to(shift1[:, None], (2, 16))

    acc1, den1, _, _ = _make_edge_kernel(split=False, row_off_mult=1)(
        h1_heads.reshape(2 * _NP, _D), src, dst, sa1, shift1_16)

    h2, sa2, sm2 = _tc_mid(acc1.reshape(2, _NP, _D),
                           den1.reshape(2, _NP, 1),
                           b1.reshape(1, 2 * _D),
                           W2[:_D], W2[_D:], A2)
    sm2 = sm2[:, 0]
    shift2 = jnp.maximum(sm2[0] + sm2[2], 0.0)
    shift2_16 = jnp.broadcast_to(shift2, (2, 16))

    acc2, den2, _, _ = _make_edge_kernel(split=True, row_off_mult=0)(
        h2, src, dst, sa2, shift2_16)

    user_rows, ctx_rows, den_ctx = _make_gather_kernel()(
        user_emb, user_idx.astype(jnp.int32),
        context_idx.reshape(-1).astype(jnp.int32), acc2, den2)

    ctx_norm = _tc_ctxnorm(ctx_rows, den_ctx.reshape(_B * _CTX, 1),
                           b2.reshape(1, _D))
    x = jnp.concatenate([user_rows, ctx_norm.reshape(_B, _CTX * _D)], axis=1)
    return _final_fc(x, Wfc, bfc)
